# baseline (device time: 115253 ns/iter reference)
import jax
import jax.numpy as jnp
from jax import lax
from jax.experimental import pallas as pl
from jax.experimental.pallas import tpu as pltpu

N_DEV = 4
EPS = 1e-5


def kernel(x, Wp):
    B, Hs, W, C = x.shape
    Cout = Wp.shape[1]
    BH = 16
    n_b = Hs // BH
    inv_n = 1.0 / float(N_DEV * Hs * W)

    def body(x_ref, w_ref, out_ref,
             stash_ref, acc_ref, st_ref, comm_ref, send_sems, recv_sems):
        h = pl.program_id(0)

        @pl.when(h < n_b)
        def _():
            xb = x_ref[...]
            stash_ref[:, pl.ds(h * BH, BH)] = xb.astype(jnp.bfloat16)
            s = jnp.sum(xb, axis=(1, 2))
            ss = jnp.sum(xb * xb, axis=(1, 2))

            @pl.when(h == 0)
            def _():
                acc_ref[:, 0, :] = s
                acc_ref[:, 1, :] = ss

            @pl.when(h > 0)
            def _():
                acc_ref[:, 0, :] += s
                acc_ref[:, 1, :] += ss

        @pl.when(h == n_b - 1)
        def _():
            my = lax.axis_index("i")
            barrier_sem = pltpu.get_barrier_semaphore()
            for k in range(1, N_DEV):
                pl.semaphore_signal(
                    barrier_sem, inc=1,
                    device_id=(lax.rem(my + k, N_DEV),),
                    device_id_type=pl.DeviceIdType.MESH,
                )
            pl.semaphore_wait(barrier_sem, N_DEV - 1)

            rdmas = []
            for k in range(1, N_DEV):
                rdma = pltpu.make_async_remote_copy(
                    src_ref=acc_ref,
                    dst_ref=comm_ref.at[k - 1],
                    send_sem=send_sems.at[k - 1],
                    recv_sem=recv_sems.at[k - 1],
                    device_id=(lax.rem(my + k, N_DEV),),
                    device_id_type=pl.DeviceIdType.MESH,
                )
                rdma.start()
                rdmas.append(rdma)
            for rdma in rdmas:
                rdma.wait_recv()
            for rdma in rdmas:
                rdma.wait_send()

            tot = acc_ref[...] + comm_ref[0] + comm_ref[1] + comm_ref[2]
            mean = tot[:, 0, :] * inv_n
            var = tot[:, 1, :] * inv_n - mean * mean
            rstd = lax.rsqrt(var + EPS)
            st_ref[0] = (mean * rstd).astype(jnp.bfloat16)
            st_ref[1] = rstd.astype(jnp.bfloat16)

        @pl.when(h >= n_b)
        def _():
            j = h - n_b
            shift = st_ref[0][:, None, None, :]
            scale = st_ref[1][:, None, None, :]
            w16 = w_ref[...].astype(jnp.bfloat16)
            xb = stash_ref[:, pl.ds(j * BH, BH)]
            hb = xb * scale - shift
            a = (hb * jax.nn.sigmoid(hb)).reshape(B * BH * W, C)
            o = jnp.dot(a, w16, preferred_element_type=jnp.float32)
            out_ref[...] = o.astype(jnp.bfloat16).reshape(B, BH, W, Cout)

    return pl.pallas_call(
        body,
        grid=(2 * n_b,),
        in_specs=[
            pl.BlockSpec((B, BH, W, C), lambda h: (0, jnp.minimum(h, n_b - 1), 0, 0)),
            pl.BlockSpec((C, Cout), lambda h: (0, 0)),
        ],
        out_specs=pl.BlockSpec(
            (B, BH, W, Cout), lambda h: (0, jnp.maximum(h - n_b, 0), 0, 0)
        ),
        out_shape=jax.ShapeDtypeStruct((B, Hs, W, Cout), jnp.bfloat16),
        scratch_shapes=[
            pltpu.VMEM((B, Hs, W, C), jnp.bfloat16),
            pltpu.VMEM((B, 2, C), jnp.float32),
            pltpu.VMEM((2, B, C), jnp.bfloat16),
            pltpu.VMEM((N_DEV - 1, B, 2, C), jnp.float32),
            pltpu.SemaphoreType.DMA((N_DEV - 1,)),
            pltpu.SemaphoreType.DMA((N_DEV - 1,)),
        ],
        compiler_params=pltpu.CompilerParams(
            collective_id=0,
            vmem_limit_bytes=100 * 1024 * 1024,
        ),
    )(x, Wp)


# device time: 73811 ns/iter; 1.5615x vs baseline; 1.5615x over previous
import jax
import jax.numpy as jnp
from jax import lax
from jax.experimental import pallas as pl
from jax.experimental.pallas import tpu as pltpu

N_DEV = 4
EPS = 1e-5


def kernel(x, Wp):
    B, Hs, W, C = x.shape
    Cout = Wp.shape[1]
    BH = 32
    n_h = Hs // BH
    inv_n = 1.0 / float(N_DEV * Hs * W)

    def stats_body(x_ref, out_ref, acc_ref, comm_ref, send_sems, recv_sems):
        h = pl.program_id(0)
        xb = x_ref[...]
        s = jnp.sum(xb, axis=(1, 2))
        ss = jnp.sum(xb * xb, axis=(1, 2))

        @pl.when(h == 0)
        def _():
            acc_ref[:, 0, :] = s
            acc_ref[:, 1, :] = ss

        @pl.when(h > 0)
        def _():
            acc_ref[:, 0, :] += s
            acc_ref[:, 1, :] += ss

        @pl.when(h == n_h - 1)
        def _():
            my = lax.axis_index("i")
            barrier_sem = pltpu.get_barrier_semaphore()
            for k in range(1, N_DEV):
                pl.semaphore_signal(
                    barrier_sem, inc=1,
                    device_id=(lax.rem(my + k, N_DEV),),
                    device_id_type=pl.DeviceIdType.MESH,
                )
            pl.semaphore_wait(barrier_sem, N_DEV - 1)

            rdmas = []
            for k in range(1, N_DEV):
                rdma = pltpu.make_async_remote_copy(
                    src_ref=acc_ref,
                    dst_ref=comm_ref.at[k - 1],
                    send_sem=send_sems.at[k - 1],
                    recv_sem=recv_sems.at[k - 1],
                    device_id=(lax.rem(my + k, N_DEV),),
                    device_id_type=pl.DeviceIdType.MESH,
                )
                rdma.start()
                rdmas.append(rdma)
            for rdma in rdmas:
                rdma.wait_recv()
            for rdma in rdmas:
                rdma.wait_send()

            tot = acc_ref[...] + comm_ref[0] + comm_ref[1] + comm_ref[2]
            mean = tot[:, 0, :] * inv_n
            var = tot[:, 1, :] * inv_n - mean * mean
            rstd = lax.rsqrt(var + EPS)
            out_ref[:, 0, :] = mean * rstd
            out_ref[:, 1, :] = rstd

    stats = pl.pallas_call(
        stats_body,
        grid=(n_h,),
        in_specs=[pl.BlockSpec((B, BH, W, C), lambda h: (0, h, 0, 0))],
        out_specs=pl.BlockSpec((B, 2, C), lambda h: (0, 0, 0)),
        out_shape=jax.ShapeDtypeStruct((B, 2, C), jnp.float32),
        scratch_shapes=[
            pltpu.VMEM((B, 2, C), jnp.float32),
            pltpu.VMEM((N_DEV - 1, B, 2, C), jnp.float32),
            pltpu.SemaphoreType.DMA((N_DEV - 1,)),
            pltpu.SemaphoreType.DMA((N_DEV - 1,)),
        ],
        compiler_params=pltpu.CompilerParams(collective_id=0),
    )(x)

    BHA = 16
    n_ha = Hs // BHA

    def apply_body(x_ref, st_ref, w_ref, out_ref):
        xb = x_ref[...].astype(jnp.bfloat16)
        shift = st_ref[:, 0, :].astype(jnp.bfloat16)[:, None, None, :]
        scale = st_ref[:, 1, :].astype(jnp.bfloat16)[:, None, None, :]
        hb = xb * scale - shift
        a = (hb * jax.nn.sigmoid(hb)).reshape(B * BHA * W, C)
        w16 = w_ref[...].astype(jnp.bfloat16)
        o = jnp.dot(a, w16, preferred_element_type=jnp.float32)
        out_ref[...] = o.astype(jnp.bfloat16).reshape(B, BHA, W, Cout)

    return pl.pallas_call(
        apply_body,
        grid=(n_ha,),
        in_specs=[
            pl.BlockSpec((B, BHA, W, C), lambda h: (0, h, 0, 0)),
            pl.BlockSpec((B, 2, C), lambda h: (0, 0, 0)),
            pl.BlockSpec((C, Cout), lambda h: (0, 0)),
        ],
        out_specs=pl.BlockSpec((B, BHA, W, Cout), lambda h: (0, h, 0, 0)),
        out_shape=jax.ShapeDtypeStruct((B, Hs, W, Cout), jnp.bfloat16),
    )(x, stats, Wp)


# device time: 72470 ns/iter; 1.5904x vs baseline; 1.0185x over previous
import jax
import jax.numpy as jnp
from jax import lax
from jax.experimental import pallas as pl
from jax.experimental.pallas import tpu as pltpu

N_DEV = 4
EPS = 1e-5


def kernel(x, Wp):
    B, Hs, W, C = x.shape
    Cout = Wp.shape[1]
    BH = 32
    n_h = Hs // BH
    inv_n = 1.0 / float(N_DEV * Hs * W)

    def stats_body(x_ref, out_ref, acc_ref, comm_ref, send_sems, recv_sems):
        h = pl.program_id(0)
        xb = x_ref[...]
        s = jnp.sum(xb, axis=(1, 2))
        ss = jnp.sum(xb * xb, axis=(1, 2))

        @pl.when(h == 0)
        def _():
            acc_ref[:, 0, :] = s
            acc_ref[:, 1, :] = ss

        @pl.when(h > 0)
        def _():
            acc_ref[:, 0, :] += s
            acc_ref[:, 1, :] += ss

        @pl.when(h == n_h - 1)
        def _():
            my = lax.axis_index("i")
            barrier_sem = pltpu.get_barrier_semaphore()
            for k in range(1, N_DEV):
                pl.semaphore_signal(
                    barrier_sem, inc=1,
                    device_id=(lax.rem(my + k, N_DEV),),
                    device_id_type=pl.DeviceIdType.MESH,
                )
            pl.semaphore_wait(barrier_sem, N_DEV - 1)

            rdmas = []
            for k in range(1, N_DEV):
                rdma = pltpu.make_async_remote_copy(
                    src_ref=acc_ref,
                    dst_ref=comm_ref.at[k - 1],
                    send_sem=send_sems.at[k - 1],
                    recv_sem=recv_sems.at[k - 1],
                    device_id=(lax.rem(my + k, N_DEV),),
                    device_id_type=pl.DeviceIdType.MESH,
                )
                rdma.start()
                rdmas.append(rdma)
            for rdma in rdmas:
                rdma.wait_recv()
            for rdma in rdmas:
                rdma.wait_send()

            tot = acc_ref[...] + comm_ref[0] + comm_ref[1] + comm_ref[2]
            mean = tot[:, 0, :] * inv_n
            var = tot[:, 1, :] * inv_n - mean * mean
            rstd = lax.rsqrt(var + EPS)
            out_ref[:, 0, :] = mean * rstd
            out_ref[:, 1, :] = rstd

    stats = pl.pallas_call(
        stats_body,
        grid=(n_h,),
        in_specs=[pl.BlockSpec((B, BH, W, C), lambda h: (0, h, 0, 0))],
        out_specs=pl.BlockSpec((B, 2, C), lambda h: (0, 0, 0)),
        out_shape=jax.ShapeDtypeStruct((B, 2, C), jnp.float32),
        scratch_shapes=[
            pltpu.VMEM((B, 2, C), jnp.float32),
            pltpu.VMEM((N_DEV - 1, B, 2, C), jnp.float32),
            pltpu.SemaphoreType.DMA((N_DEV - 1,)),
            pltpu.SemaphoreType.DMA((N_DEV - 1,)),
        ],
        compiler_params=pltpu.CompilerParams(
            collective_id=0, vmem_limit_bytes=100 * 1024 * 1024
        ),
    )(x)

    BHA = 32
    n_ha = Hs // BHA

    def apply_body(x_ref, st_ref, w_ref, out_ref):
        xb = x_ref[...].astype(jnp.bfloat16)
        shift = st_ref[:, 0, :].astype(jnp.bfloat16)[:, None, None, :]
        scale = st_ref[:, 1, :].astype(jnp.bfloat16)[:, None, None, :]
        hb = xb * scale - shift
        sig = 0.5 * jnp.tanh(0.5 * hb) + 0.5
        a = (hb * sig).reshape(B * BHA * W, C)
        w16 = w_ref[...].astype(jnp.bfloat16)
        o = jnp.dot(a, w16, preferred_element_type=jnp.float32)
        out_ref[...] = o.astype(jnp.bfloat16).reshape(B, BHA, W, Cout)

    return pl.pallas_call(
        apply_body,
        grid=(n_ha,),
        in_specs=[
            pl.BlockSpec((B, BHA, W, C), lambda h: (0, h, 0, 0)),
            pl.BlockSpec((B, 2, C), lambda h: (0, 0, 0)),
            pl.BlockSpec((C, Cout), lambda h: (0, 0)),
        ],
        out_specs=pl.BlockSpec((B, BHA, W, Cout), lambda h: (0, h, 0, 0)),
        out_shape=jax.ShapeDtypeStruct((B, Hs, W, Cout), jnp.bfloat16),
        compiler_params=pltpu.CompilerParams(
            vmem_limit_bytes=100 * 1024 * 1024
        ),
    )(x, stats, Wp)
